# baseline (speedup 1.0000x reference)
import jax
import jax.numpy as jnp
from jax import lax
from jax.experimental import pallas as pl
from jax.experimental.pallas import tpu as pltpu
from jax.experimental.pallas import tpu_sc as plsc

_B = 4
_L = 2048
_LABEL_NUM = 4096
_LO = 0.1 / _LABEL_NUM
_HI = 0.9 + 0.1 / _LABEL_NUM

_NC = 2
_NS = 16
_NW = _NC * _NS

_ROWS = _B * _L
_RPW = _ROWS // _NW
_WPB = _L // _RPW
_K = 8
_CHUNKS = _RPW // _K
_NBUF = 3


def _body(labels_hbm, out_hbm, idx_v, buf0_v, buf1_v, buf2_v, sem0, sem1, sem2, lsem):
    wid = lax.axis_index("s") * _NC + lax.axis_index("c")
    b = wid // _WPB
    l0 = (wid % _WPB) * _RPW

    lcp = pltpu.async_copy(
        labels_hbm.at[b].at[pl.ds(l0, _RPW)], idx_v.at[pl.ds(0, _RPW)], lsem
    )

    lo_vec = jnp.full((16,), _LO, jnp.float32)
    hi_vec = jnp.full((16,), _HI, jnp.float32)

    def fill_block(buf):
        def body(j, carry):
            r = j // (_LABEL_NUM // 256)
            coff = (j % (_LABEL_NUM // 256)) * 256
            for u in range(16):
                buf[r, pl.ds(coff + u * 16, 16)] = lo_vec
            return carry
        lax.fori_loop(0, _K * _LABEL_NUM // 256, body, 0)

    lanes = lax.iota(jnp.int32, 16)
    rows8 = lanes & (_K - 1)
    mask8 = lanes < _K

    bufs = (buf0_v, buf1_v, buf2_v)
    sems = (sem0, sem1, sem2)

    def chunk_cols(c):
        return idx_v[pl.ds(c * _K, 16)]

    def start(c, p):
        cols = chunk_cols(c)
        plsc.store_scatter(bufs[p], [rows8, cols], hi_vec, mask=mask8)
        return pltpu.async_copy(
            bufs[p], out_hbm.at[b].at[pl.ds(l0 + c * _K, _K)], sems[p]
        )

    fill_block(bufs[0])
    lcp.wait()
    idx_v[pl.ds(_RPW, 16)] = jnp.zeros((16,), jnp.int32)
    start(0, 0)
    fill_block(bufs[1])
    start(1, 1)
    fill_block(bufs[2])
    start(2, 2)

    def loop(i, carry):
        for p in range(_NBUF):
            c = _NBUF * i + p
            pltpu.make_async_copy(
                bufs[p], out_hbm.at[b].at[pl.ds(l0 + (c - _NBUF) * _K, _K)], sems[p]
            ).wait()
            old_cols = chunk_cols(c - _NBUF)
            plsc.store_scatter(bufs[p], [rows8, old_cols], lo_vec, mask=mask8)
            cols = chunk_cols(c)
            plsc.store_scatter(bufs[p], [rows8, cols], hi_vec, mask=mask8)
            pltpu.async_copy(
                bufs[p], out_hbm.at[b].at[pl.ds(l0 + c * _K, _K)], sems[p]
            )
        return carry

    lax.fori_loop(1, _CHUNKS // _NBUF, loop, 0)
    for c in range(_CHUNKS - _CHUNKS % _NBUF, _CHUNKS):
        p = c % _NBUF
        pltpu.make_async_copy(
            bufs[p], out_hbm.at[b].at[pl.ds(l0 + (c - _NBUF) * _K, _K)], sems[p]
        ).wait()
        old_cols = chunk_cols(c - _NBUF)
        plsc.store_scatter(bufs[p], [rows8, old_cols], lo_vec, mask=mask8)
        cols = chunk_cols(c)
        plsc.store_scatter(bufs[p], [rows8, cols], hi_vec, mask=mask8)
        pltpu.async_copy(bufs[p], out_hbm.at[b].at[pl.ds(l0 + c * _K, _K)], sems[p])
    for c in range(_CHUNKS - _NBUF, _CHUNKS):
        p = c % _NBUF
        pltpu.make_async_copy(
            bufs[p], out_hbm.at[b].at[pl.ds(l0 + c * _K, _K)], sems[p]
        ).wait()


@jax.jit
def kernel(Input):
    mesh = plsc.VectorSubcoreMesh(core_axis_name="c", subcore_axis_name="s")
    out = pl.kernel(
        _body,
        out_type=jax.ShapeDtypeStruct((_B, _L, _LABEL_NUM), jnp.float32),
        mesh=mesh,
        scratch_types=[
            pltpu.VMEM((_RPW + 16,), jnp.int32),
            pltpu.VMEM((_K, _LABEL_NUM), jnp.float32),
            pltpu.VMEM((_K, _LABEL_NUM), jnp.float32),
            pltpu.VMEM((_K, _LABEL_NUM), jnp.float32),
            pltpu.SemaphoreType.DMA,
            pltpu.SemaphoreType.DMA,
            pltpu.SemaphoreType.DMA,
            pltpu.SemaphoreType.DMA,
        ],
        compiler_params=pltpu.CompilerParams(needs_layout_passes=False),
    )(Input)
    return out


# speedup vs baseline: 1.0055x; 1.0055x over previous
import jax
import jax.numpy as jnp
from jax import lax
from jax.experimental import pallas as pl
from jax.experimental.pallas import tpu as pltpu
from jax.experimental.pallas import tpu_sc as plsc

_B = 4
_L = 2048
_LABEL_NUM = 4096
_LO = 0.1 / _LABEL_NUM
_HI = 0.9 + 0.1 / _LABEL_NUM

_NC = 2
_NS = 16
_NW = _NC * _NS

_ROWS = _B * _L
_RPW = _ROWS // _NW
_WPB = _L // _RPW
_K = 8
_CHUNKS = _RPW // _K


def _body(labels_hbm, out_hbm, idx_v, buf0_v, buf1_v, sem0, sem1, lsem):
    wid = lax.axis_index("s") * _NC + lax.axis_index("c")
    b = wid // _WPB
    l0 = (wid % _WPB) * _RPW

    lcp = pltpu.async_copy(
        labels_hbm.at[b].at[pl.ds(l0, _RPW)], idx_v.at[pl.ds(0, _RPW)], lsem
    )

    lo_vec = jnp.full((16,), _LO, jnp.float32)
    hi_vec = jnp.full((16,), _HI, jnp.float32)

    def fill_block(buf):
        def body(j, carry):
            r = j // (_LABEL_NUM // 256)
            coff = (j % (_LABEL_NUM // 256)) * 256
            for u in range(16):
                buf[r, pl.ds(coff + u * 16, 16)] = lo_vec
            return carry
        lax.fori_loop(0, _K * _LABEL_NUM // 256, body, 0)

    lanes = lax.iota(jnp.int32, 16)
    rows8 = lanes & (_K - 1)
    mask8 = lanes < _K

    bufs = (buf0_v, buf1_v)
    sems = (sem0, sem1)

    def chunk_cols(c):
        return idx_v[pl.ds(c * _K, 16)]

    def start(c, p):
        cols = chunk_cols(c)
        plsc.store_scatter(bufs[p], [rows8, cols], hi_vec, mask=mask8)
        return pltpu.async_copy(
            bufs[p], out_hbm.at[b].at[pl.ds(l0 + c * _K, _K)], sems[p]
        )

    fill_block(buf0_v)
    lcp.wait()
    idx_v[pl.ds(_RPW, 16)] = jnp.zeros((16,), jnp.int32)
    cp0 = start(0, 0)
    fill_block(buf1_v)
    cp1 = start(1, 1)

    def loop(i, carry):
        for p in range(2):
            c = 2 * i + p
            pltpu.make_async_copy(
                bufs[p], out_hbm.at[b].at[pl.ds(l0 + (c - 2) * _K, _K)], sems[p]
            ).wait()
            old_cols = chunk_cols(c - 2)
            plsc.store_scatter(bufs[p], [rows8, old_cols], lo_vec, mask=mask8)
            cols = chunk_cols(c)
            plsc.store_scatter(bufs[p], [rows8, cols], hi_vec, mask=mask8)
            pltpu.async_copy(
                bufs[p], out_hbm.at[b].at[pl.ds(l0 + c * _K, _K)], sems[p]
            )
        return carry

    lax.fori_loop(1, _CHUNKS // 2, loop, 0)
    cp0.wait()
    cp1.wait()


@jax.jit
def kernel(Input):
    mesh = plsc.VectorSubcoreMesh(core_axis_name="c", subcore_axis_name="s")
    out = pl.kernel(
        _body,
        out_type=jax.ShapeDtypeStruct((_B, _L, _LABEL_NUM), jnp.float32),
        mesh=mesh,
        scratch_types=[
            pltpu.VMEM((_RPW + 16,), jnp.int32),
            pltpu.VMEM((_K, _LABEL_NUM), jnp.float32),
            pltpu.VMEM((_K, _LABEL_NUM), jnp.float32),
            pltpu.SemaphoreType.DMA,
            pltpu.SemaphoreType.DMA,
            pltpu.SemaphoreType.DMA,
        ],
        compiler_params=pltpu.CompilerParams(needs_layout_passes=False),
    )(Input)
    return out
